# baseline (device time: 25449 ns/iter reference)
import jax
import jax.numpy as jnp
from jax import lax
from jax.experimental import pallas as pl
from jax.experimental.pallas import tpu as pltpu

N_DEV = 8
LINKS = (1, 3, 4)
PEERS = (1, 3, 4, 6)
Q = 4
SCALE = 5.0


def kernel(A, B):
    m_per, k = A.shape
    n = B.shape[1]
    qr = m_per // Q
    hr = m_per // 2

    def body(a_ref, b_ref, out_ref, comm_ref, send_sems, recv_sems):
        my = lax.axis_index("i")

        def rdma(g, m, q, dst=None):
            dst = (m ^ g) if dst is None else dst
            return pltpu.make_async_remote_copy(
                src_ref=comm_ref.at[m, pl.ds(q * qr, qr)],
                dst_ref=comm_ref.at[dst, pl.ds(q * qr, qr)],
                send_sem=send_sems.at[dst, q],
                recv_sem=recv_sems.at[dst, q],
                device_id=(my ^ g,),
                device_id_type=pl.DeviceIdType.MESH,
            )

        def wait_recv(slot, q):
            pltpu.make_async_remote_copy(
                src_ref=comm_ref.at[slot, pl.ds(q * qr, qr)],
                dst_ref=comm_ref.at[slot, pl.ds(q * qr, qr)],
                send_sem=send_sems.at[slot, q],
                recv_sem=recv_sems.at[slot, q],
                device_id=(my,),
                device_id_type=pl.DeviceIdType.MESH,
            ).wait_recv()

        def compute_half(mask, h):
            origin = my ^ mask
            out_ref[pl.ds(origin * m_per + h * hr, hr), :] = jnp.dot(
                comm_ref[mask, pl.ds(h * hr, hr)].astype(jnp.bfloat16),
                b_scaled,
                preferred_element_type=jnp.float32,
            ).astype(jnp.bfloat16)

        barrier_sem = pltpu.get_barrier_semaphore()
        for g in PEERS:
            pl.semaphore_signal(
                barrier_sem, inc=1,
                device_id=(my ^ g,), device_id_type=pl.DeviceIdType.MESH,
            )
        pl.semaphore_wait(barrier_sem, len(PEERS))

        comm_ref[0] = jnp.clip(
            jnp.round(a_ref[...] * (127.0 / SCALE)), -127.0, 127.0
        ).astype(jnp.int8)

        sends = []

        def start(g, m, q, dst=None):
            r = rdma(g, m, q, dst)
            r.start()
            sends.append(r)

        for q in range(Q):
            for g in PEERS:
                start(g, 0, q)

        b_scaled = (b_ref[...] * (SCALE / 127.0)).astype(jnp.bfloat16)
        compute_half(0, 0)
        compute_half(0, 1)

        ph2 = {1: (4, 1), 3: (1, 3), 4: (3, 4)}
        for q in range(Q):
            for mask in (1, 3, 4):
                wait_recv(mask, q)
                g, m = ph2[mask]
                start(g, m, q)
            if q == 1:
                compute_half(1, 0)
                compute_half(3, 0)
            elif q == 2:
                compute_half(4, 0)

        compute_half(1, 1)
        compute_half(3, 1)
        compute_half(4, 1)

        wait_recv(6, 0)
        wait_recv(6, 1)
        compute_half(6, 0)
        wait_recv(7, 0)
        wait_recv(7, 1)
        compute_half(7, 0)
        wait_recv(5, 0)
        wait_recv(5, 1)
        compute_half(5, 0)
        wait_recv(2, 0)
        wait_recv(2, 1)
        compute_half(2, 0)
        wait_recv(6, 2)
        wait_recv(6, 3)
        compute_half(6, 1)
        wait_recv(7, 2)
        wait_recv(7, 3)
        compute_half(7, 1)
        wait_recv(5, 2)
        wait_recv(5, 3)
        compute_half(5, 1)
        wait_recv(2, 2)
        wait_recv(2, 3)
        compute_half(2, 1)

        for r in sends:
            r.wait_send()

    return pl.pallas_call(
        body,
        out_shape=jax.ShapeDtypeStruct((N_DEV * m_per, n), jnp.bfloat16),
        in_specs=[
            pl.BlockSpec(memory_space=pltpu.VMEM),
            pl.BlockSpec(memory_space=pltpu.VMEM),
        ],
        out_specs=pl.BlockSpec(memory_space=pltpu.VMEM),
        scratch_shapes=[
            pltpu.VMEM((N_DEV, m_per, k), jnp.int8),
            pltpu.SemaphoreType.DMA((N_DEV, Q)),
            pltpu.SemaphoreType.DMA((N_DEV, Q)),
        ],
        compiler_params=pltpu.CompilerParams(collective_id=0),
    )(A, B)


# device time: 22094 ns/iter; 1.1519x vs baseline; 1.1519x over previous
import jax
import jax.numpy as jnp
from jax import lax
from jax.experimental import pallas as pl
from jax.experimental.pallas import tpu as pltpu

N_DEV = 8
LINKS = (1, 3, 4)
Q = 4
SCALE = 5.0


def kernel(A, B):
    m_per, k = A.shape
    n = B.shape[1]
    qr = m_per // Q
    hr = m_per // 2

    def body(a_ref, b_ref, out_ref, comm_ref, send_sems, recv_sems):
        my = lax.axis_index("i")

        def rdma(g, m, q, dst=None):
            dst = (m ^ g) if dst is None else dst
            return pltpu.make_async_remote_copy(
                src_ref=comm_ref.at[m, pl.ds(q * qr, qr)],
                dst_ref=comm_ref.at[dst, pl.ds(q * qr, qr)],
                send_sem=send_sems.at[dst, q],
                recv_sem=recv_sems.at[dst, q],
                device_id=(my ^ g,),
                device_id_type=pl.DeviceIdType.MESH,
            )

        def wait_recv(slot, q):
            pltpu.make_async_remote_copy(
                src_ref=comm_ref.at[slot, pl.ds(q * qr, qr)],
                dst_ref=comm_ref.at[slot, pl.ds(q * qr, qr)],
                send_sem=send_sems.at[slot, q],
                recv_sem=recv_sems.at[slot, q],
                device_id=(my,),
                device_id_type=pl.DeviceIdType.MESH,
            ).wait_recv()

        def compute_half(mask, h):
            origin = my ^ mask
            out_ref[pl.ds(origin * m_per + h * hr, hr), :] = jnp.dot(
                comm_ref[mask, pl.ds(h * hr, hr)].astype(jnp.bfloat16),
                b_scaled,
                preferred_element_type=jnp.float32,
            ).astype(jnp.bfloat16)

        barrier_sem = pltpu.get_barrier_semaphore()
        for g in LINKS:
            pl.semaphore_signal(
                barrier_sem, inc=1,
                device_id=(my ^ g,), device_id_type=pl.DeviceIdType.MESH,
            )
        pl.semaphore_wait(barrier_sem, len(LINKS))

        comm_ref[0] = jnp.clip(
            jnp.round(a_ref[...] * (127.0 / SCALE)), -127.0, 127.0
        ).astype(jnp.int8)

        sends = []

        def start(g, m, q, dst=None):
            r = rdma(g, m, q, dst)
            r.start()
            sends.append(r)

        for q in range(Q):
            for g in LINKS:
                start(g, 0, q)

        b_scaled = (b_ref[...] * (SCALE / 127.0)).astype(jnp.bfloat16)
        compute_half(0, 0)
        compute_half(0, 1)

        ph2 = {1: (4, 1), 3: (1, 3), 4: (3, 4)}
        for q in range(Q):
            for mask in (1, 3, 4):
                wait_recv(mask, q)
                g, m = ph2[mask]
                start(g, m, q)
            if q == 1:
                compute_half(1, 0)
                compute_half(3, 0)
            elif q == 2:
                compute_half(4, 0)

        compute_half(1, 1)
        wait_recv(7, 0)
        start(1, 7, 0, dst=6)
        compute_half(3, 1)
        wait_recv(7, 1)
        start(1, 7, 1, dst=6)
        compute_half(4, 1)
        compute_half(7, 0)
        wait_recv(5, 2)
        start(3, 5, 2, dst=6)
        wait_recv(2, 0)
        wait_recv(2, 1)
        compute_half(2, 0)
        wait_recv(5, 3)
        start(3, 5, 3, dst=6)

        wait_recv(5, 0)
        wait_recv(5, 1)
        compute_half(5, 0)
        wait_recv(7, 2)
        wait_recv(7, 3)
        compute_half(7, 1)
        wait_recv(2, 2)
        wait_recv(2, 3)
        compute_half(2, 1)
        compute_half(5, 1)

        wait_recv(6, 0)
        wait_recv(6, 1)
        compute_half(6, 0)
        wait_recv(6, 2)
        wait_recv(6, 3)
        compute_half(6, 1)

        for r in sends:
            r.wait_send()

    return pl.pallas_call(
        body,
        out_shape=jax.ShapeDtypeStruct((N_DEV * m_per, n), jnp.bfloat16),
        in_specs=[
            pl.BlockSpec(memory_space=pltpu.VMEM),
            pl.BlockSpec(memory_space=pltpu.VMEM),
        ],
        out_specs=pl.BlockSpec(memory_space=pltpu.VMEM),
        scratch_shapes=[
            pltpu.VMEM((N_DEV, m_per, k), jnp.int8),
            pltpu.SemaphoreType.DMA((N_DEV, Q)),
            pltpu.SemaphoreType.DMA((N_DEV, Q)),
        ],
        compiler_params=pltpu.CompilerParams(collective_id=0),
    )(A, B)
